# Initial kernel scaffold; baseline (speedup 1.0000x reference)
import jax
import jax.numpy as jnp
from jax.experimental import pallas as pl


def kernel(scores, k):
    raise NotImplementedError("write your pallas kernel here")


# speedup vs baseline: 18.4321x; 18.4321x over previous
import jax
import jax.numpy as jnp
from jax.experimental import pallas as pl
from jax.experimental.pallas import tpu as pltpu

_ROWS_PER_BLOCK = 8


def _prune_body(kk_ref, x_ref, o_ref):
    x = x_ref[...]
    u = jax.lax.bitcast_convert_type(jnp.abs(x), jnp.int32)
    kk = kk_ref[0]

    def step(i, t):
        b = 30 - i
        cand = t | (1 << b)
        cnt = jnp.sum((u < cand).astype(jnp.int32), axis=1, keepdims=True)
        return jnp.where(cnt < kk, cand, t)

    t0 = jnp.zeros((x.shape[0], 1), jnp.int32)
    t = jax.lax.fori_loop(0, 31, step, t0)
    t = jnp.where(kk > 0, t, -1)
    o_ref[...] = jnp.where(u <= t, 0.0, x)


def kernel(scores, k):
    r, c = scores.shape
    kk = jnp.clip(k, 0, c // 10).astype(jnp.int32).reshape(1)
    return pl.pallas_call(
        _prune_body,
        grid=(r // _ROWS_PER_BLOCK,),
        in_specs=[
            pl.BlockSpec(memory_space=pltpu.SMEM),
            pl.BlockSpec((_ROWS_PER_BLOCK, c), lambda i: (i, 0)),
        ],
        out_specs=pl.BlockSpec((_ROWS_PER_BLOCK, c), lambda i: (i, 0)),
        out_shape=jax.ShapeDtypeStruct((r, c), jnp.float32),
    )(kk, scores)
